# Initial kernel scaffold; baseline (speedup 1.0000x reference)
import jax
import jax.numpy as jnp
from jax.experimental import pallas as pl


def kernel(x, edge_index, W0, b0, W1, b1):
    raise NotImplementedError("write your pallas kernel here")


# speedup vs baseline: 8.6315x; 8.6315x over previous
import functools

import jax
import jax.numpy as jnp
import numpy as np
from jax import lax
from jax.experimental import pallas as pl
from jax.experimental.pallas import tpu as pltpu
from jax.experimental.pallas import tpu_sc as plsc

MIN_NORM = 1e-15
EPS = 1e-5

N, D, E = 10000, 128, 320000
DP = 144
NC, NS = 2, 16
NW = NC * NS
EPW = E // NW
K = 125
CHUNKS = EPW // K
NA = 10240
RPT = NA // NS
ZROWS = 128
BLK = 400


def _artanh(x):
    x = jnp.clip(x, -1.0 + 1e-7, 1.0 - 1e-7)
    return 0.5 * jnp.log((1.0 + x) / (1.0 - x))


def _norm(x):
    return jnp.clip(jnp.sqrt(jnp.sum(x * x, axis=-1, keepdims=True)), MIN_NORM, None)


def _proj(x):
    norm = _norm(x)
    maxnorm = 1.0 - EPS
    return jnp.where(norm > maxnorm, x / norm * maxnorm, x)


def _expmap0(u):
    u_norm = _norm(u)
    return jnp.tanh(u_norm) * u / u_norm


def _logmap0(p):
    p_norm = _norm(p)
    return p / p_norm * _artanh(p_norm)


def _mobius_add(x, y):
    x2 = jnp.sum(x * x, axis=-1, keepdims=True)
    y2 = jnp.sum(y * y, axis=-1, keepdims=True)
    xy = jnp.sum(x * y, axis=-1, keepdims=True)
    num = (1.0 + 2.0 * xy + y2) * x + (1.0 - x2) * y
    denom = 1.0 + 2.0 * xy + x2 * y2
    return num / jnp.clip(denom, MIN_NORM, None)


def _layer_pre_body(first, h_ref, wt_ref, hb_ref, out_ref):
    h = h_ref[...]
    if first:
        h = _proj(_expmap0(h))
    x_norm = _norm(h)
    mx = jnp.dot(h, wt_ref[...], preferred_element_type=jnp.float32)
    mx_norm = _norm(mx)
    res = jnp.tanh(mx_norm / x_norm * _artanh(x_norm)) * mx / mx_norm
    zero_mask = jnp.max(jnp.abs(mx), axis=-1, keepdims=True) == 0.0
    res = jnp.where(zero_mask, 0.0, res)
    res = _proj(res)
    hb = hb_ref[...][:1, :]
    res = _proj(_mobius_add(res, hb))
    xt = _logmap0(res)
    lane16 = lax.broadcasted_iota(jnp.int32, (BLK, DP - D), 1)
    pad = jnp.where(lane16 == 0, 1.0, 0.0)
    out_ref[...] = jnp.concatenate([xt, pad], axis=1)


def _layer_pre(h, wt, hb, first):
    return pl.pallas_call(
        functools.partial(_layer_pre_body, first),
        grid=(N // BLK,),
        in_specs=[
            pl.BlockSpec((BLK, D), lambda i: (i, 0)),
            pl.BlockSpec((D, D), lambda i: (0, 0)),
            pl.BlockSpec((8, D), lambda i: (0, 0)),
        ],
        out_specs=pl.BlockSpec((BLK, DP), lambda i: (i, 0)),
        out_shape=jax.ShapeDtypeStruct((N, DP), jnp.float32),
    )(h, wt, hb)


def _sc_agg_body(xt_hbm, src_hbm, dst_hbm, zeros_hbm, out_hbm,
                 acc, src_v, dst_v, rowbuf, sem):
    c = lax.axis_index("c")
    s = lax.axis_index("s")
    wid = s * NC + c
    pltpu.sync_copy(src_hbm.at[wid], src_v)
    pltpu.sync_copy(dst_hbm.at[wid], dst_v)
    pltpu.sync_copy(zeros_hbm, rowbuf)
    row0 = s * RPT
    for k in range(RPT // ZROWS):
        pltpu.sync_copy(rowbuf, acc.at[pl.ds(row0 + k * ZROWS, ZROWS)])
    plsc.subcore_barrier()

    @pl.loop(0, CHUNKS)
    def _chunk(j):
        pltpu.async_copy(xt_hbm.at[src_v.at[j]], rowbuf.at[pl.ds(0, K)], sem).wait()
        pltpu.sync_copy(rowbuf.at[pl.ds(0, K)], acc.at[dst_v.at[j]], add=True)

    plsc.subcore_barrier()
    for k in range(RPT // ZROWS):
        sl = pl.ds(row0 + k * ZROWS, ZROWS)
        pltpu.sync_copy(acc.at[sl], out_hbm.at[c, sl])


_sc_agg = pl.kernel(
    _sc_agg_body,
    out_type=jax.ShapeDtypeStruct((NC, NA, DP), jnp.float32),
    mesh=plsc.VectorSubcoreMesh(core_axis_name="c", subcore_axis_name="s"),
    compiler_params=pltpu.CompilerParams(use_tc_tiling_on_sc=False),
    scratch_types=[
        pltpu.VMEM_SHARED((NA, DP), jnp.float32),
        pltpu.VMEM((CHUNKS, K), jnp.int32),
        pltpu.VMEM((CHUNKS, K), jnp.int32),
        pltpu.VMEM((ZROWS, DP), jnp.float32),
        pltpu.SemaphoreType.DMA,
    ],
)


def _layer_post_body(acc0_ref, acc1_ref, out_ref):
    acc = acc0_ref[...] + acc1_ref[...]
    lane = lax.broadcasted_iota(jnp.int32, (BLK, DP), 1)
    deg = jnp.sum(jnp.where(lane == D, acc, 0.0), axis=-1, keepdims=True)
    deg = jnp.clip(deg, 1.0, None)
    support = acc[:, :D] / deg
    h = _proj(_expmap0(support))
    ht = jnp.maximum(_logmap0(h), 0.0)
    out_ref[...] = _proj(_expmap0(ht))


def _layer_post(acc0, acc1):
    return pl.pallas_call(
        _layer_post_body,
        grid=(N // BLK,),
        in_specs=[
            pl.BlockSpec((BLK, DP), lambda i: (i, 0)),
            pl.BlockSpec((BLK, DP), lambda i: (i, 0)),
        ],
        out_specs=pl.BlockSpec((BLK, D), lambda i: (i, 0)),
        out_shape=jax.ShapeDtypeStruct((N, D), jnp.float32),
    )(acc0, acc1)


def _hyp_bias(b):
    hb = _proj(_expmap0(b.reshape(1, -1)))
    return jnp.broadcast_to(hb, (8, D))


def kernel(x, edge_index, W0, b0, W1, b1):
    src3 = edge_index[0].reshape(NW, CHUNKS, K)
    dst3 = edge_index[1].reshape(NW, CHUNKS, K)
    zeros = jnp.zeros((ZROWS, DP), jnp.float32)
    hb0 = _hyp_bias(b0)
    hb1 = _hyp_bias(b1)

    xtp1 = _layer_pre(x, W0.T, hb0, first=True)
    accs1 = _sc_agg(xtp1, src3, dst3, zeros)
    h1 = _layer_post(accs1[0, :N], accs1[1, :N])

    xtp2 = _layer_pre(h1, W1.T, hb1, first=False)
    accs2 = _sc_agg(xtp2, src3, dst3, zeros)
    h2 = _layer_post(accs2[0, :N], accs2[1, :N])

    return jnp.stack([h1, h2])
